# baseline (speedup 1.0000x reference)
import functools

import jax
import jax.numpy as jnp
from jax import lax
from jax.experimental import pallas as pl
from jax.experimental.pallas import tpu as pltpu
from jax.experimental.pallas import tpu_sc as plsc

NC = 2
NS = 16
NW = NC * NS

DW = 128
DO = 128
EB = 64
NB = 4


def _make_edge_pass(n_acc, n_chunk_rows):
    rpw = n_chunk_rows // NW
    nh = 4
    rph = rpw // nh
    stripe = n_acc // NS

    mesh = plsc.VectorSubcoreMesh(core_axis_name="c", subcore_axis_name="s")

    @functools.partial(
        pl.kernel,
        out_type=[jax.ShapeDtypeStruct((NC, n_acc, DO), jnp.float32)],
        mesh=mesh,
        scratch_types=[
            pltpu.VMEM((rph, EB), jnp.int32),
            pltpu.VMEM((rph, EB), jnp.int32),
            pltpu.VMEM((NB, EB, DW), jnp.float32),
            pltpu.SemaphoreType.DMA((NB,)),
            pltpu.SemaphoreType.DMA((NB,)),
            pltpu.VMEM_SHARED((n_acc, DW), jnp.float32),
        ],
    )
    def edge_pass(table, src, dst, zrows, acc_out,
                  src_v, dst_v, rows, gsem, ssem, acc_sh):
        c = lax.axis_index("c")
        s = lax.axis_index("s")
        wid = s * NC + c
        st = s * stripe

        pltpu.sync_copy(zrows, rows.at[0])
        for k in range(stripe // EB):
            pltpu.sync_copy(rows.at[0], acc_sh.at[pl.ds(st + k * EB, EB)])
        rem = stripe % EB
        if rem:
            pltpu.sync_copy(rows.at[0, pl.ds(0, rem)],
                            acc_sh.at[pl.ds(st + (stripe // EB) * EB, rem)])
        plsc.subcore_barrier()

        row0 = wid * rpw

        def gfire(b, q):
            pltpu.async_copy(table.at[src_v.at[b]], rows.at[q], gsem.at[q])

        def gwait(b, q):
            pltpu.make_async_copy(table.at[src_v.at[b]], rows.at[q],
                                  gsem.at[q]).wait()

        def sfire(b, q):
            pltpu.async_copy(rows.at[q], acc_sh.at[dst_v.at[b]], ssem.at[q],
                             add=True)

        def swait(b, q):
            pltpu.make_async_copy(rows.at[q], acc_sh.at[dst_v.at[b]],
                                  ssem.at[q]).wait()

        def half(hh, carry):
            base = row0 + hh * rph
            pltpu.sync_copy(src.at[pl.ds(base, rph)], src_v)
            pltpu.sync_copy(dst.at[pl.ds(base, rph)], dst_v)
            gfire(0, 0)
            gfire(1, 1)

            def quad(it, c2):
                b = it * 4
                pl.when(it > 0)(lambda: swait(b - 2, 2))
                gfire(b + 2, 2)
                gwait(b, 0)
                sfire(b, 0)
                pl.when(it > 0)(lambda: swait(b - 1, 3))
                gfire(b + 3, 3)
                gwait(b + 1, 1)
                sfire(b + 1, 1)
                swait(b, 0)
                pl.when(b + 4 < rph)(lambda: gfire(b + 4, 0))
                gwait(b + 2, 2)
                sfire(b + 2, 2)
                swait(b + 1, 1)
                pl.when(b + 5 < rph)(lambda: gfire(b + 5, 1))
                gwait(b + 3, 3)
                sfire(b + 3, 3)
                return c2

            lax.fori_loop(0, rph // 4, quad, 0)
            swait(rph - 2, 2)
            swait(rph - 1, 3)
            return carry

        lax.fori_loop(0, nh, half, 0)

        plsc.subcore_barrier()
        pltpu.sync_copy(acc_sh.at[pl.ds(st, stripe)],
                        acc_out.at[c, pl.ds(st, stripe)])

    return edge_pass


def _layer1_pre(x, w1, d_hid, bm):
    n, k = x.shape
    dw2 = 2 * d_hid

    def body(x_ref, w_ref, t_ref, r_ref):
        z = jnp.dot(x_ref[...], w_ref[...], preferred_element_type=jnp.float32)
        ones = jnp.ones((bm, 1), jnp.float32)
        zeros = jnp.zeros((bm, DW - d_hid - 1), jnp.float32)
        t_ref[...] = jnp.concatenate([z[:, :d_hid], ones, zeros], axis=1)
        r_ref[...] = z[:, d_hid:]

    return pl.pallas_call(
        body,
        grid=(n // bm,),
        in_specs=[pl.BlockSpec((bm, k), lambda i: (i, 0)),
                  pl.BlockSpec((k, dw2), lambda i: (0, 0))],
        out_specs=[pl.BlockSpec((bm, DW), lambda i: (i, 0)),
                   pl.BlockSpec((bm, d_hid), lambda i: (i, 0))],
        out_shape=[jax.ShapeDtypeStruct((n, DW), jnp.float32),
                   jax.ShapeDtypeStruct((n, d_hid), jnp.float32)],
    )(x, w1)


def _layer1_post(acc1, r1, b1, bm):
    n, d = r1.shape

    def body(a_ref, r1_ref, b1_ref, h128_ref, h_ref, idg_ref):
        a = a_ref[0] + a_ref[1]
        inv = 1.0 / jnp.maximum(a[:, d:d + 1], 1.0)
        h = jnp.maximum(a[:, :d] * inv + r1_ref[...] + b1_ref[...], 0.0)
        h128_ref[...] = jnp.concatenate(
            [h, jnp.zeros((bm, DW - d), jnp.float32)], axis=1)
        h_ref[...] = h
        idg_ref[...] = jnp.broadcast_to(inv, (bm, 8))

    return pl.pallas_call(
        body,
        grid=(n // bm,),
        in_specs=[pl.BlockSpec((2, bm, DO), lambda i: (0, i, 0)),
                  pl.BlockSpec((bm, d), lambda i: (i, 0)),
                  pl.BlockSpec((1, d), lambda i: (0, 0))],
        out_specs=[pl.BlockSpec((bm, DW), lambda i: (i, 0)),
                   pl.BlockSpec((bm, d), lambda i: (i, 0)),
                   pl.BlockSpec((bm, 8), lambda i: (i, 0))],
        out_shape=[jax.ShapeDtypeStruct((n, DW), jnp.float32),
                   jax.ShapeDtypeStruct((n, d), jnp.float32),
                   jax.ShapeDtypeStruct((n, 8), jnp.float32)],
    )(acc1, r1, b1)


def _layer2_post(acc2, h, idg, w_l, w_r, b2, bm):
    d, m = w_l.shape
    n = h.shape[0]

    def body(a2_ref, h_ref, idg_ref, wl_ref, wr_ref, b2_ref, o_ref):
        mean = (a2_ref[0] + a2_ref[1])[:, :d] * idg_ref[:, :1]
        o_ref[...] = (jnp.dot(mean, wl_ref[...],
                              preferred_element_type=jnp.float32)
                      + jnp.dot(h_ref[...], wr_ref[...],
                                preferred_element_type=jnp.float32)
                      + b2_ref[...])

    return pl.pallas_call(
        body,
        grid=(n // bm,),
        in_specs=[pl.BlockSpec((2, bm, DO), lambda i: (0, i, 0)),
                  pl.BlockSpec((bm, d), lambda i: (i, 0)),
                  pl.BlockSpec((bm, 8), lambda i: (i, 0)),
                  pl.BlockSpec((d, m), lambda i: (0, 0)),
                  pl.BlockSpec((d, m), lambda i: (0, 0)),
                  pl.BlockSpec((1, m), lambda i: (0, 0))],
        out_specs=pl.BlockSpec((bm, m), lambda i: (i, 0)),
        out_shape=jax.ShapeDtypeStruct((n, m), jnp.float32),
    )(acc2, h, idg, w_l, w_r, b2)


def kernel(x, edge_index, W1_l, W1_r, b1, W2_l, W2_r, b2):
    n, d_in = x.shape
    d_hid = W1_l.shape[1]
    d_out = W2_l.shape[1]
    e = edge_index.shape[1]

    rpw = -(-e // (NW * EB))
    rpw = -(-rpw // 16) * 16
    n_chunk_rows = NW * rpw
    e_pad = n_chunk_rows * EB

    n_acc = -(-(n + 1) // (NS * 8)) * (NS * 8)

    src = edge_index[0].astype(jnp.int32)
    dst = edge_index[1].astype(jnp.int32)
    pad_i = jnp.arange(e_pad - e, dtype=jnp.int32)
    src_p = jnp.concatenate([src, pad_i % n]).reshape(n_chunk_rows, EB)
    dst_p = jnp.concatenate(
        [dst, n + pad_i % (n_acc - n)]).reshape(n_chunk_rows, EB)

    zrows = jnp.zeros((EB, DW), jnp.float32)

    w1 = jnp.concatenate([W1_l, W1_r], axis=1)
    table1, r1 = _layer1_pre(x, w1, d_hid, bm=1000)

    edge_pass = _make_edge_pass(n_acc, n_chunk_rows)
    (acc1,) = edge_pass(table1, src_p, dst_p, zrows)

    h128, h16, idg = _layer1_post(acc1, r1, b1.reshape(1, d_hid), bm=1000)

    (acc2,) = edge_pass(h128, src_p, dst_p, zrows)

    out = _layer2_post(acc2, h16, idg,
                       W2_l, W2_r, b2.reshape(1, d_out), bm=1000)
    return out


# speedup vs baseline: 1.0199x; 1.0199x over previous
import functools

import jax
import jax.numpy as jnp
from jax import lax
from jax.experimental import pallas as pl
from jax.experimental.pallas import tpu as pltpu
from jax.experimental.pallas import tpu_sc as plsc

NC = 2
NS = 16
NW = NC * NS

DW = 128
DO = 128
EB = 128


def _make_edge_pass(n_acc, n_chunk_rows):
    rpw = n_chunk_rows // NW
    nh = 2
    rph = rpw // nh
    stripe = n_acc // NS

    mesh = plsc.VectorSubcoreMesh(core_axis_name="c", subcore_axis_name="s")

    @functools.partial(
        pl.kernel,
        out_type=[jax.ShapeDtypeStruct((NC, n_acc, DO), jnp.float32)],
        mesh=mesh,
        scratch_types=[
            pltpu.VMEM((rph, EB), jnp.int32),
            pltpu.VMEM((rph, EB), jnp.int32),
            pltpu.VMEM((EB, DW), jnp.float32),
            pltpu.VMEM((EB, DW), jnp.float32),
            pltpu.SemaphoreType.DMA,
            pltpu.SemaphoreType.DMA,
            pltpu.VMEM_SHARED((n_acc, DW), jnp.float32),
        ],
    )
    def edge_pass(table, src, dst, zrows, acc_out,
                  src_v, dst_v, rows0, rows1, g0, g1, acc_sh):
        c = lax.axis_index("c")
        s = lax.axis_index("s")
        wid = s * NC + c
        st = s * stripe

        pltpu.sync_copy(zrows, rows0)
        for k in range(stripe // EB):
            pltpu.sync_copy(rows0, acc_sh.at[pl.ds(st + k * EB, EB)])
        rem = stripe % EB
        if rem:
            pltpu.sync_copy(rows0.at[pl.ds(0, rem)],
                            acc_sh.at[pl.ds(st + (stripe // EB) * EB, rem)])
        plsc.subcore_barrier()

        row0 = wid * rpw

        def half(hh, carry):
            base = row0 + hh * rph
            pltpu.sync_copy(src.at[pl.ds(base, rph)], src_v)
            pltpu.sync_copy(dst.at[pl.ds(base, rph)], dst_v)
            pltpu.async_copy(table.at[src_v.at[0]], rows0, g0)

            def pair(it, c2):
                b0 = it * 2
                pltpu.async_copy(table.at[src_v.at[b0 + 1]], rows1, g1)
                pltpu.make_async_copy(table.at[src_v.at[b0]], rows0, g0).wait()
                pltpu.sync_copy(rows0, acc_sh.at[dst_v.at[b0]], add=True)

                @pl.when(b0 + 2 < rph)
                def _fire_next():
                    pltpu.async_copy(table.at[src_v.at[b0 + 2]], rows0, g0)

                pltpu.make_async_copy(table.at[src_v.at[b0 + 1]],
                                      rows1, g1).wait()
                pltpu.sync_copy(rows1, acc_sh.at[dst_v.at[b0 + 1]], add=True)
                return c2

            lax.fori_loop(0, rph // 2, pair, 0)
            return carry

        lax.fori_loop(0, nh, half, 0)

        plsc.subcore_barrier()
        pltpu.sync_copy(acc_sh.at[pl.ds(st, stripe)],
                        acc_out.at[c, pl.ds(st, stripe)])

    return edge_pass


def _layer1_pre(x, w1, d_hid, bm):
    n, k = x.shape
    dw2 = 2 * d_hid

    def body(x_ref, w_ref, t_ref, r_ref):
        z = jnp.dot(x_ref[...], w_ref[...], preferred_element_type=jnp.float32)
        ones = jnp.ones((bm, 1), jnp.float32)
        zeros = jnp.zeros((bm, DW - d_hid - 1), jnp.float32)
        t_ref[...] = jnp.concatenate([z[:, :d_hid], ones, zeros], axis=1)
        r_ref[...] = z[:, d_hid:]

    return pl.pallas_call(
        body,
        grid=(n // bm,),
        in_specs=[pl.BlockSpec((bm, k), lambda i: (i, 0)),
                  pl.BlockSpec((k, dw2), lambda i: (0, 0))],
        out_specs=[pl.BlockSpec((bm, DW), lambda i: (i, 0)),
                   pl.BlockSpec((bm, d_hid), lambda i: (i, 0))],
        out_shape=[jax.ShapeDtypeStruct((n, DW), jnp.float32),
                   jax.ShapeDtypeStruct((n, d_hid), jnp.float32)],
    )(x, w1)


def _layer1_post(acc1, r1, b1, bm):
    n, d = r1.shape

    def body(a_ref, r1_ref, b1_ref, h128_ref, h_ref, idg_ref):
        a = a_ref[0] + a_ref[1]
        inv = 1.0 / jnp.maximum(a[:, d:d + 1], 1.0)
        h = jnp.maximum(a[:, :d] * inv + r1_ref[...] + b1_ref[...], 0.0)
        h128_ref[...] = jnp.concatenate(
            [h, jnp.zeros((bm, DW - d), jnp.float32)], axis=1)
        h_ref[...] = h
        idg_ref[...] = jnp.broadcast_to(inv, (bm, 8))

    return pl.pallas_call(
        body,
        grid=(n // bm,),
        in_specs=[pl.BlockSpec((2, bm, DO), lambda i: (0, i, 0)),
                  pl.BlockSpec((bm, d), lambda i: (i, 0)),
                  pl.BlockSpec((1, d), lambda i: (0, 0))],
        out_specs=[pl.BlockSpec((bm, DW), lambda i: (i, 0)),
                   pl.BlockSpec((bm, d), lambda i: (i, 0)),
                   pl.BlockSpec((bm, 8), lambda i: (i, 0))],
        out_shape=[jax.ShapeDtypeStruct((n, DW), jnp.float32),
                   jax.ShapeDtypeStruct((n, d), jnp.float32),
                   jax.ShapeDtypeStruct((n, 8), jnp.float32)],
    )(acc1, r1, b1)


def _layer2_post(acc2, h, idg, w_l, w_r, b2, bm):
    d, m = w_l.shape
    n = h.shape[0]

    def body(a2_ref, h_ref, idg_ref, wl_ref, wr_ref, b2_ref, o_ref):
        mean = (a2_ref[0] + a2_ref[1])[:, :d] * idg_ref[:, :1]
        o_ref[...] = (jnp.dot(mean, wl_ref[...],
                              preferred_element_type=jnp.float32)
                      + jnp.dot(h_ref[...], wr_ref[...],
                                preferred_element_type=jnp.float32)
                      + b2_ref[...])

    return pl.pallas_call(
        body,
        grid=(n // bm,),
        in_specs=[pl.BlockSpec((2, bm, DO), lambda i: (0, i, 0)),
                  pl.BlockSpec((bm, d), lambda i: (i, 0)),
                  pl.BlockSpec((bm, 8), lambda i: (i, 0)),
                  pl.BlockSpec((d, m), lambda i: (0, 0)),
                  pl.BlockSpec((d, m), lambda i: (0, 0)),
                  pl.BlockSpec((1, m), lambda i: (0, 0))],
        out_specs=pl.BlockSpec((bm, m), lambda i: (i, 0)),
        out_shape=jax.ShapeDtypeStruct((n, m), jnp.float32),
    )(acc2, h, idg, w_l, w_r, b2)


def kernel(x, edge_index, W1_l, W1_r, b1, W2_l, W2_r, b2):
    n, d_in = x.shape
    d_hid = W1_l.shape[1]
    d_out = W2_l.shape[1]
    e = edge_index.shape[1]

    rpw = -(-e // (NW * EB))
    rpw = -(-rpw // 4) * 4
    n_chunk_rows = NW * rpw
    e_pad = n_chunk_rows * EB

    n_acc = -(-(n + 1) // (NS * 8)) * (NS * 8)

    src = edge_index[0].astype(jnp.int32)
    dst = edge_index[1].astype(jnp.int32)
    pad_i = jnp.arange(e_pad - e, dtype=jnp.int32)
    src_p = jnp.concatenate([src, pad_i % n]).reshape(n_chunk_rows, EB)
    dst_p = jnp.concatenate(
        [dst, n + pad_i % (n_acc - n)]).reshape(n_chunk_rows, EB)

    zrows = jnp.zeros((EB, DW), jnp.float32)

    w1 = jnp.concatenate([W1_l, W1_r], axis=1)
    table1, r1 = _layer1_pre(x, w1, d_hid, bm=1000)

    edge_pass = _make_edge_pass(n_acc, n_chunk_rows)
    (acc1,) = edge_pass(table1, src_p, dst_p, zrows)

    h128, h16, idg = _layer1_post(acc1, r1, b1.reshape(1, d_hid), bm=1000)

    (acc2,) = edge_pass(h128, src_p, dst_p, zrows)

    out = _layer2_post(acc2, h16, idg,
                       W2_l, W2_r, b2.reshape(1, d_out), bm=1000)
    return out
